# baseline (speedup 1.0000x reference)
import functools

import jax
import jax.numpy as jnp
from jax.experimental import pallas as pl
from jax.experimental.pallas import tpu as pltpu

_C = 3
_HID = 64
_W = 64
_PIX2 = 2 * _W * _W


def _shift(v, k):
    ng = v.shape[1]
    if k % ng == 0:
        return v
    return pltpu.roll(v, (-k) % ng, 1)


def _admm_body(y_ref, w1_ref, w2_ref, b1_ref, b2_ref, sp_ref, o_ref,
               st_ref, yb_ref, mh_ref, p1_ref, h_ref, y2_ref,
               *, npair, nb):
    ng = npair * _PIX2

    for p in range(npair):
        yb_ref[0:_C, p * _PIX2:(p + 1) * _PIX2] = y_ref[p]
    st_ref[0:_C] = yb_ref[0:_C]
    st_ref[4:4 + _C] = jnp.zeros((_C, ng), jnp.float32)

    col = jax.lax.broadcasted_iota(jnp.int32, (4, _PIX2), 1) & (_W - 1)
    mh_ref[0:4] = (col >= 1).astype(jnp.float32)
    mh_ref[4:8] = (col <= _W - 2).astype(jnp.float32)

    def mh(v, ddx):
        if ddx == 0:
            return v
        base = 0 if ddx < 0 else 4
        return v * mh_ref[base:base + 4]

    taps = [(ddy, ddx) for ddy in (-1, 0, 1) for ddx in (-1, 0, 1)]

    def tap_row(ddy, ddx):
        return 4 * ((ddy + 1) * 3 + (ddx + 1))

    for blk in range(nb):
        sigma = sp_ref[0, blk]
        pen = sp_ref[0, nb + blk]

        for p in range(npair):
            chunk = pl.ds(p * _PIX2, _PIX2)
            stc = st_ref[:, chunk]
            xv = jnp.concatenate(
                [stc[0:_C] + stc[4:4 + _C],
                 jnp.full((1, _PIX2), sigma, jnp.float32)], axis=0)
            for ddx in (-1, 0, 1):
                xs = mh(_shift(xv, ddx), ddx)
                for ddy in (-1, 0, 1):
                    p1_ref[pl.ds(tap_row(ddy, ddx), 4), chunk] = (
                        _shift(xs, 128 * ddy))
        for ddy, ddx in taps:
            if ddy == 0:
                continue
            r = tap_row(ddy, ddx)
            for p in range(npair):
                edge = p * _PIX2 + (0 if ddy < 0 else _PIX2 - 128)
                p1_ref[pl.ds(r, 4), pl.ds(edge, 128)] = (
                    jnp.zeros((4, 128), jnp.float32))

        h = jnp.dot(w1_ref[...], p1_ref[0:36],
                    preferred_element_type=jnp.float32)
        h_ref[...] = jnp.maximum(h + b1_ref[...], 0.0).astype(jnp.bfloat16)

        y2_ref[0:36] = jnp.dot(w2_ref[...], h_ref[...],
                               preferred_element_type=jnp.float32)
        for ddy, ddx in taps:
            if ddy == 0:
                continue
            r = tap_row(ddy, ddx)
            for p in range(npair):
                edge = p * _PIX2 + (_PIX2 - 128 if ddy < 0 else 0)
                y2_ref[pl.ds(r, 4), pl.ds(edge, 128)] = (
                    jnp.zeros((4, 128), jnp.float32))

        inv = 1.0 / (1.0 + pen)
        for p in range(npair):
            chunk = pl.ds(p * _PIX2, _PIX2)
            dacc = None
            for ddx in (-1, 0, 1):
                inner = None
                for ddy in (-1, 0, 1):
                    v = _shift(y2_ref[pl.ds(tap_row(ddy, ddx), 4), chunk],
                               128 * ddy)
                    inner = v if inner is None else inner + v
                s = mh(_shift(inner, ddx), ddx)
                dacc = s if dacc is None else dacc + s
            d = dacc[0:_C] + b2_ref[...]
            u = st_ref[pl.ds(4, _C), chunk]
            xnew = (yb_ref[0:_C, chunk] + pen * (d - u)) * inv
            st_ref[0:_C, chunk] = xnew
            st_ref[pl.ds(4, _C), chunk] = u + xnew - d

    for p in range(npair):
        o_ref[p] = st_ref[0:_C, p * _PIX2:(p + 1) * _PIX2]


def kernel(y, w1, b1, w2, b2, penaltys, sigmas):
    b, c, hh, ww = y.shape
    nb = penaltys.shape[0]
    g = 16 if b % 16 == 0 else 2
    npair = g // 2
    ng = npair * _PIX2

    yi = y.reshape(b // 2, 2, c, hh, ww).transpose(0, 2, 3, 1, 4)
    yi = yi.reshape(b // 2, c, _PIX2)

    w1t = w1.reshape(9 * (c + 1), _HID).T
    w2p = jnp.pad(w2.transpose(0, 1, 3, 2),
                  ((0, 0), (0, 0), (0, 1), (0, 0))).reshape(36, _HID)
    w2p = w2p.astype(jnp.bfloat16)
    b1c = b1.reshape(_HID, 1)
    b2c = b2.reshape(c, 1)
    sp = jnp.concatenate([sigmas, penaltys]).reshape(1, 2 * nb)

    body = functools.partial(_admm_body, npair=npair, nb=nb)
    out = pl.pallas_call(
        body,
        grid=(b // g,),
        in_specs=[
            pl.BlockSpec((npair, c, _PIX2), lambda i: (i, 0, 0)),
            pl.BlockSpec((_HID, 36), lambda i: (0, 0)),
            pl.BlockSpec((36, _HID), lambda i: (0, 0)),
            pl.BlockSpec((_HID, 1), lambda i: (0, 0)),
            pl.BlockSpec((c, 1), lambda i: (0, 0)),
            pl.BlockSpec(memory_space=pltpu.SMEM),
        ],
        out_specs=pl.BlockSpec((npair, c, _PIX2), lambda i: (i, 0, 0)),
        out_shape=jax.ShapeDtypeStruct((b // 2, c, _PIX2), jnp.float32),
        scratch_shapes=[
            pltpu.VMEM((8, ng), jnp.float32),
            pltpu.VMEM((4, ng), jnp.float32),
            pltpu.VMEM((8, _PIX2), jnp.float32),
            pltpu.VMEM((40, ng), jnp.float32),
            pltpu.VMEM((_HID, ng), jnp.bfloat16),
            pltpu.VMEM((40, ng), jnp.float32),
        ],
        compiler_params=pltpu.CompilerParams(
            dimension_semantics=("parallel",)),
    )(yi, w1t, w2p, b1c, b2c, sp)

    out = out.reshape(b // 2, c, hh, 2, ww).transpose(0, 3, 1, 2, 4)
    return out.reshape(b, c, hh, ww)


# speedup vs baseline: 1.0517x; 1.0517x over previous
import functools

import jax
import jax.numpy as jnp
from jax.experimental import pallas as pl
from jax.experimental.pallas import tpu as pltpu

_C = 3
_HID = 64
_W = 64
_PIX2 = 2 * _W * _W


def _shift(v, k):
    ng = v.shape[1]
    if k % ng == 0:
        return v
    return pltpu.roll(v, (-k) % ng, 1)


def _admm_body(y_ref, w1_ref, w2_ref, b1_ref, b2_ref, sp_ref, o_ref,
               st_ref, yb_ref, mh_ref, p1_ref, h_ref, y2_ref,
               *, npair, nb):
    ng = npair * _PIX2

    for p in range(npair):
        yb_ref[0:_C, p * _PIX2:(p + 1) * _PIX2] = y_ref[p]
    st_ref[0:_C] = yb_ref[0:_C]
    st_ref[4:4 + _C] = jnp.zeros((_C, ng), jnp.float32)

    col = jax.lax.broadcasted_iota(jnp.int32, (4, _PIX2), 1) & (_W - 1)
    mh_ref[0:4] = (col >= 1).astype(jnp.float32)
    mh_ref[4:8] = (col <= _W - 2).astype(jnp.float32)

    def mh(v, ddx):
        if ddx == 0:
            return v
        base = 0 if ddx < 0 else 4
        return v * mh_ref[base:base + 4]

    taps = [(ddy, ddx) for ddy in (-1, 0, 1) for ddx in (-1, 0, 1)]

    def tap_row(ddy, ddx):
        return 4 * ((ddy + 1) * 3 + (ddx + 1))

    for blk in range(nb):
        sigma = sp_ref[0, blk]
        pen = sp_ref[0, nb + blk]

        for p in range(npair):
            chunk = pl.ds(p * _PIX2, _PIX2)
            stc = st_ref[:, chunk]
            xv = jnp.concatenate(
                [stc[0:_C] + stc[4:4 + _C],
                 jnp.full((1, _PIX2), sigma, jnp.float32)], axis=0)
            for ddx in (-1, 0, 1):
                xs = mh(_shift(xv, ddx), ddx)
                for ddy in (-1, 0, 1):
                    p1_ref[pl.ds(tap_row(ddy, ddx), 4), chunk] = (
                        _shift(xs, 128 * ddy))
        for ddy, ddx in taps:
            if ddy == 0:
                continue
            r = tap_row(ddy, ddx)
            for p in range(npair):
                edge = p * _PIX2 + (0 if ddy < 0 else _PIX2 - 128)
                p1_ref[pl.ds(r, 4), pl.ds(edge, 128)] = (
                    jnp.zeros((4, 128), jnp.float32))

        h = jnp.dot(w1_ref[...], p1_ref[0:36],
                    preferred_element_type=jnp.float32)
        h_ref[...] = jnp.maximum(h + b1_ref[...], 0.0)

        y2_ref[0:36] = jnp.dot(w2_ref[...], h_ref[...],
                               preferred_element_type=jnp.float32)
        for ddy, ddx in taps:
            if ddy == 0:
                continue
            r = tap_row(ddy, ddx)
            for p in range(npair):
                edge = p * _PIX2 + (_PIX2 - 128 if ddy < 0 else 0)
                y2_ref[pl.ds(r, 4), pl.ds(edge, 128)] = (
                    jnp.zeros((4, 128), jnp.float32))

        inv = 1.0 / (1.0 + pen)
        for p in range(npair):
            chunk = pl.ds(p * _PIX2, _PIX2)
            dacc = None
            for ddx in (-1, 0, 1):
                inner = None
                for ddy in (-1, 0, 1):
                    v = _shift(y2_ref[pl.ds(tap_row(ddy, ddx), 4), chunk],
                               128 * ddy)
                    inner = v if inner is None else inner + v
                s = mh(_shift(inner, ddx), ddx)
                dacc = s if dacc is None else dacc + s
            d = dacc[0:_C] + b2_ref[...]
            u = st_ref[pl.ds(4, _C), chunk]
            xnew = (yb_ref[0:_C, chunk] + pen * (d - u)) * inv
            st_ref[0:_C, chunk] = xnew
            st_ref[pl.ds(4, _C), chunk] = u + xnew - d

    for p in range(npair):
        o_ref[p] = st_ref[0:_C, p * _PIX2:(p + 1) * _PIX2]


def kernel(y, w1, b1, w2, b2, penaltys, sigmas):
    b, c, hh, ww = y.shape
    nb = penaltys.shape[0]
    g = 16 if b % 16 == 0 else 2
    npair = g // 2
    ng = npair * _PIX2

    yi = y.reshape(b // 2, 2, c, hh, ww).transpose(0, 2, 3, 1, 4)
    yi = yi.reshape(b // 2, c, _PIX2)

    w1t = w1.reshape(9 * (c + 1), _HID).T
    w2p = jnp.pad(w2.transpose(0, 1, 3, 2),
                  ((0, 0), (0, 0), (0, 1), (0, 0))).reshape(36, _HID)
    b1c = b1.reshape(_HID, 1)
    b2c = b2.reshape(c, 1)
    sp = jnp.concatenate([sigmas, penaltys]).reshape(1, 2 * nb)

    body = functools.partial(_admm_body, npair=npair, nb=nb)
    out = pl.pallas_call(
        body,
        grid=(b // g,),
        in_specs=[
            pl.BlockSpec((npair, c, _PIX2), lambda i: (i, 0, 0)),
            pl.BlockSpec((_HID, 36), lambda i: (0, 0)),
            pl.BlockSpec((36, _HID), lambda i: (0, 0)),
            pl.BlockSpec((_HID, 1), lambda i: (0, 0)),
            pl.BlockSpec((c, 1), lambda i: (0, 0)),
            pl.BlockSpec(memory_space=pltpu.SMEM),
        ],
        out_specs=pl.BlockSpec((npair, c, _PIX2), lambda i: (i, 0, 0)),
        out_shape=jax.ShapeDtypeStruct((b // 2, c, _PIX2), jnp.float32),
        scratch_shapes=[
            pltpu.VMEM((8, ng), jnp.float32),
            pltpu.VMEM((4, ng), jnp.float32),
            pltpu.VMEM((8, _PIX2), jnp.float32),
            pltpu.VMEM((40, ng), jnp.float32),
            pltpu.VMEM((_HID, ng), jnp.float32),
            pltpu.VMEM((40, ng), jnp.float32),
        ],
        compiler_params=pltpu.CompilerParams(
            dimension_semantics=("parallel",)),
    )(yi, w1t, w2p, b1c, b2c, sp)

    out = out.reshape(b // 2, c, hh, 2, ww).transpose(0, 3, 1, 2, 4)
    return out.reshape(b, c, hh, ww)
